# baseline (speedup 1.0000x reference)
import functools

import jax
import jax.numpy as jnp
from jax.experimental import pallas as pl
from jax.experimental.pallas import tpu as pltpu

H = 16384
K = 50
DECAY = 0.95
SUB = 8
LANE = H // SUB
UBLK = 1024


def _matmul_kernel(x_ref, w_ref, out_ref):
    out_ref[...] = jax.lax.dot_general(
        x_ref[...], w_ref[...],
        (((1,), (1,)), ((), ())),
        preferred_element_type=jnp.float32,
    )


def _scan_kernel(enc_ref, out_ref, inhib_ref):
    sub = jax.lax.broadcasted_iota(jnp.int32, (SUB, LANE), 0)
    lane = jax.lax.broadcasted_iota(jnp.int32, (SUB, LANE), 1)
    lin = sub * LANE + lane
    inhib_ref[...] = jnp.zeros((SUB, LANE), jnp.float32)
    nrows = enc_ref.shape[0]

    def row_step(i, _):
        enc = enc_ref[i]
        inhib = inhib_ref[...]
        refr = jnp.abs(enc) * (1.0 - inhib)

        def extract(_, carry):
            refr_c, fired = carry
            m = jnp.max(refr_c)
            cand = jnp.where(refr_c == m, lin, jnp.int32(1 << 30))
            first = jnp.min(cand)
            hit = lin == first
            fired = jnp.where(hit, 1.0, fired)
            refr_c = jnp.where(hit, -jnp.inf, refr_c)
            return refr_c, fired

        _, fired = jax.lax.fori_loop(
            0, K, extract, (refr, jnp.zeros((SUB, LANE), jnp.float32)))

        filtered = enc * fired
        inhib_ref[...] = inhib * DECAY + fired

        pos = filtered > 0.0
        p = jnp.sum(pos.astype(jnp.int32))
        need = K - p
        zmask = filtered == 0.0
        bits = jax.lax.bitcast_convert_type(filtered, jnp.int32)
        ordv = jnp.where(bits == 0, lin, lin + H)

        def bisect(_, lohi):
            lo, hi = lohi
            mid = (lo + hi) // 2
            c = jnp.sum((zmask & (ordv < mid)).astype(jnp.int32))
            ge = c >= need
            return jnp.where(ge, lo, mid), jnp.where(ge, mid, hi)

        _, t = jax.lax.fori_loop(
            0, 15, bisect, (jnp.int32(0), jnp.int32(2 * H)))
        zsel = zmask & (ordv < t) & (need > 0)
        out_ref[i] = jnp.where(pos | zsel, 1.0, 0.0)
        return 0

    jax.lax.fori_loop(0, nrows, row_step, 0)


def kernel(inputs, W):
    B = inputs.shape[0]
    x = inputs.reshape(B, -1)

    encoding = pl.pallas_call(
        _matmul_kernel,
        grid=(H // UBLK,),
        in_specs=[
            pl.BlockSpec((B, x.shape[1]), lambda i: (0, 0)),
            pl.BlockSpec((UBLK, x.shape[1]), lambda i: (i, 0)),
        ],
        out_specs=pl.BlockSpec((B, UBLK), lambda i: (0, i)),
        out_shape=jax.ShapeDtypeStruct((B, H), jnp.float32),
    )(x, W)

    enc3 = encoding.reshape(B, SUB, LANE)
    mask3 = pl.pallas_call(
        _scan_kernel,
        in_specs=[pl.BlockSpec(memory_space=pltpu.VMEM)],
        out_specs=pl.BlockSpec(memory_space=pltpu.VMEM),
        out_shape=jax.ShapeDtypeStruct((B, SUB, LANE), jnp.float32),
        scratch_shapes=[pltpu.VMEM((SUB, LANE), jnp.float32)],
    )(enc3)
    return mask3.reshape(B, H)


# speedup vs baseline: 4.7100x; 4.7100x over previous
import functools

import jax
import jax.numpy as jnp
from jax.experimental import pallas as pl
from jax.experimental.pallas import tpu as pltpu

H = 16384
K = 50
DECAY = 0.95
SUB = 8
LANE = H // SUB
UBLK = 1024


def _matmul_kernel(x_ref, w_ref, out_ref):
    out_ref[...] = jax.lax.dot_general(
        x_ref[...], w_ref[...],
        (((1,), (1,)), ((), ())),
        preferred_element_type=jnp.float32,
    )


def _scan_kernel(enc_ref, out_ref, inhib_ref):
    sub = jax.lax.broadcasted_iota(jnp.int32, (SUB, LANE), 0)
    lane = jax.lax.broadcasted_iota(jnp.int32, (SUB, LANE), 1)
    lin = sub * LANE + lane
    inhib_ref[...] = jnp.zeros((SUB, LANE), jnp.float32)
    nrows = enc_ref.shape[0]

    def row_step(i, _):
        enc = enc_ref[i]
        inhib = inhib_ref[...]
        refr = jnp.abs(enc) * (1.0 - inhib)

        u = jax.lax.bitcast_convert_type(refr, jnp.int32)

        def vbisect(_, lohi):
            lo, hi = lohi
            s = (hi - lo + 7) // 8
            new_lo, new_hi = lo, hi
            ges = []
            for j in range(1, 8):
                m = jnp.minimum(lo + s * j, hi)
                c = jnp.sum((u >= m).astype(jnp.float32))
                ges.append((m, c >= jnp.float32(K)))
            for m, ge in ges:
                new_lo = jnp.where(ge, m, new_lo)
            for m, ge in reversed(ges):
                new_hi = jnp.where(ge, new_hi, m)
            return new_lo, new_hi

        t, _ = jax.lax.fori_loop(
            0, 11, vbisect,
            (jnp.int32(0), jnp.int32(0x7F800000)))

        gt = u > t
        eq = u == t
        c_at = jnp.sum((u >= t).astype(jnp.int32))
        cgt = jnp.sum(gt.astype(jnp.int32))

        def take_one(carry):
            sel, left = carry
            cand = jnp.where(eq & (sel == 0.0), lin, jnp.int32(1 << 30))
            first = jnp.min(cand)
            return jnp.where(lin == first, 1.0, sel), left - 1

        left0 = jnp.where(c_at > K, K - cgt, jnp.int32(0))
        sel, _ = jax.lax.while_loop(
            lambda carry: carry[1] > 0, take_one,
            (jnp.zeros((SUB, LANE), jnp.float32), left0))

        fired = jnp.where(
            c_at > K,
            jnp.where(gt, 1.0, sel),
            (u >= t).astype(jnp.float32))

        inhib_ref[...] = inhib * DECAY + fired
        out_ref[i] = fired
        return 0

    jax.lax.fori_loop(0, nrows, row_step, 0)


RB = 8


def _final_kernel(enc_ref, fired_ref, out_ref):
    enc = enc_ref[...]
    fired = fired_ref[...]
    sub = jax.lax.broadcasted_iota(jnp.int32, (RB, SUB, LANE), 1)
    lane = jax.lax.broadcasted_iota(jnp.int32, (RB, SUB, LANE), 2)
    lin = sub * LANE + lane
    pos = (fired > 0.0) & (enc > 0.0)
    p = jnp.sum(jnp.sum(pos.astype(jnp.float32), axis=2, keepdims=True),
                axis=1, keepdims=True)
    need = jnp.float32(K) - p
    zmask = (fired == 0.0) | (enc == 0.0)
    encbits = jax.lax.bitcast_convert_type(enc, jnp.int32)
    ordv = jnp.where(encbits >= 0, lin, lin + H)

    def bisect(_, lohi):
        lo, hi = lohi
        s = (hi - lo + 7) // 8
        new_lo, new_hi = lo, hi
        ges = []
        for j in range(1, 8):
            m = jnp.minimum(lo + s * j, hi)
            cnt = (zmask & (ordv < m)).astype(jnp.float32)
            c = jnp.sum(jnp.sum(cnt, axis=2, keepdims=True),
                        axis=1, keepdims=True)
            ges.append((m, c >= need))
        for m, ge in ges:
            new_hi = jnp.where(ge, jnp.minimum(new_hi, m), new_hi)
        for m, ge in ges:
            new_lo = jnp.where(ge, new_lo, jnp.maximum(new_lo, m))
        return new_lo, new_hi

    z = jnp.zeros((RB, 1, 1), jnp.int32)
    _, t2 = jax.lax.fori_loop(0, 6, bisect, (z, z + 2 * H))
    zsel = zmask & (ordv < t2) & (need > 0.0)
    out_ref[...] = jnp.where(pos | zsel, 1.0, 0.0)


def kernel(inputs, W):
    B = inputs.shape[0]
    x = inputs.reshape(B, -1)

    encoding = pl.pallas_call(
        _matmul_kernel,
        grid=(H // UBLK,),
        in_specs=[
            pl.BlockSpec((B, x.shape[1]), lambda i: (0, 0)),
            pl.BlockSpec((UBLK, x.shape[1]), lambda i: (i, 0)),
        ],
        out_specs=pl.BlockSpec((B, UBLK), lambda i: (0, i)),
        out_shape=jax.ShapeDtypeStruct((B, H), jnp.float32),
    )(x, W)

    enc3 = encoding.reshape(B, SUB, LANE)
    fired3 = pl.pallas_call(
        _scan_kernel,
        in_specs=[pl.BlockSpec(memory_space=pltpu.VMEM)],
        out_specs=pl.BlockSpec(memory_space=pltpu.VMEM),
        out_shape=jax.ShapeDtypeStruct((B, SUB, LANE), jnp.float32),
        scratch_shapes=[pltpu.VMEM((SUB, LANE), jnp.float32)],
    )(enc3)
    mask3 = pl.pallas_call(
        _final_kernel,
        grid=(B // RB,),
        in_specs=[
            pl.BlockSpec((RB, SUB, LANE), lambda i: (i, 0, 0)),
            pl.BlockSpec((RB, SUB, LANE), lambda i: (i, 0, 0)),
        ],
        out_specs=pl.BlockSpec((RB, SUB, LANE), lambda i: (i, 0, 0)),
        out_shape=jax.ShapeDtypeStruct((B, SUB, LANE), jnp.float32),
    )(enc3, fired3)
    return mask3.reshape(B, H)


# speedup vs baseline: 4.7811x; 1.0151x over previous
import functools

import jax
import jax.numpy as jnp
from jax.experimental import pallas as pl
from jax.experimental.pallas import tpu as pltpu

H = 16384
K = 50
DECAY = 0.95
SUB = 8
LANE = H // SUB
UBLK = 1024


def _matmul_kernel(x_ref, w_ref, out_ref):
    out_ref[...] = jax.lax.dot_general(
        x_ref[...], w_ref[...],
        (((1,), (1,)), ((), ())),
        preferred_element_type=jnp.float32,
    )


def _scan_kernel(enc_ref, out_ref, inhib_ref):
    sub = jax.lax.broadcasted_iota(jnp.int32, (SUB, LANE), 0)
    lane = jax.lax.broadcasted_iota(jnp.int32, (SUB, LANE), 1)
    lin = sub * LANE + lane
    inhib_ref[...] = jnp.zeros((SUB, LANE), jnp.float32)
    nrows = enc_ref.shape[0]

    def row_step(i, _):
        enc = enc_ref[i]
        inhib = inhib_ref[...]
        refr = jnp.abs(enc) * (1.0 - inhib)

        u = jax.lax.bitcast_convert_type(refr, jnp.int32)

        def vbisect(_, lohi):
            lo, hi = lohi
            s = (hi - lo + 15) // 16
            new_lo, new_hi = lo, hi
            ges = []
            for j in range(1, 16):
                m = jnp.minimum(lo + s * j, hi)
                c = jnp.sum((u >= m).astype(jnp.float32))
                ges.append((m, c >= jnp.float32(K)))
            for m, ge in ges:
                new_lo = jnp.where(ge, m, new_lo)
            for m, ge in reversed(ges):
                new_hi = jnp.where(ge, new_hi, m)
            return new_lo, new_hi

        t, _ = jax.lax.fori_loop(
            0, 8, vbisect,
            (jnp.int32(0), jnp.int32(0x7F800000)))

        gt = u > t
        eq = u == t
        c_at = jnp.sum((u >= t).astype(jnp.int32))
        cgt = jnp.sum(gt.astype(jnp.int32))

        def take_one(carry):
            sel, left = carry
            cand = jnp.where(eq & (sel == 0.0), lin, jnp.int32(1 << 30))
            first = jnp.min(cand)
            return jnp.where(lin == first, 1.0, sel), left - 1

        left0 = jnp.where(c_at > K, K - cgt, jnp.int32(0))
        sel, _ = jax.lax.while_loop(
            lambda carry: carry[1] > 0, take_one,
            (jnp.zeros((SUB, LANE), jnp.float32), left0))

        fired = jnp.where(
            c_at > K,
            jnp.where(gt, 1.0, sel),
            (u >= t).astype(jnp.float32))

        inhib_ref[...] = inhib * DECAY + fired
        out_ref[i] = fired
        return 0

    jax.lax.fori_loop(0, nrows, row_step, 0)


RB = 8


def _final_kernel(enc_ref, fired_ref, out_ref):
    enc = enc_ref[...]
    fired = fired_ref[...]
    sub = jax.lax.broadcasted_iota(jnp.int32, (RB, SUB, LANE), 1)
    lane = jax.lax.broadcasted_iota(jnp.int32, (RB, SUB, LANE), 2)
    lin = sub * LANE + lane
    pos = (fired > 0.0) & (enc > 0.0)
    p = jnp.sum(jnp.sum(pos.astype(jnp.float32), axis=2, keepdims=True),
                axis=1, keepdims=True)
    need = jnp.float32(K) - p
    zmask = (fired == 0.0) | (enc == 0.0)
    encbits = jax.lax.bitcast_convert_type(enc, jnp.int32)
    ordv = jnp.where(encbits >= 0, lin, lin + H)

    def bisect(_, lohi):
        lo, hi = lohi
        s = (hi - lo + 7) // 8
        new_lo, new_hi = lo, hi
        ges = []
        for j in range(1, 8):
            m = jnp.minimum(lo + s * j, hi)
            cnt = (zmask & (ordv < m)).astype(jnp.float32)
            c = jnp.sum(jnp.sum(cnt, axis=2, keepdims=True),
                        axis=1, keepdims=True)
            ges.append((m, c >= need))
        for m, ge in ges:
            new_hi = jnp.where(ge, jnp.minimum(new_hi, m), new_hi)
        for m, ge in ges:
            new_lo = jnp.where(ge, new_lo, jnp.maximum(new_lo, m))
        return new_lo, new_hi

    z = jnp.zeros((RB, 1, 1), jnp.int32)
    _, t2 = jax.lax.fori_loop(0, 6, bisect, (z, z + 2 * H))
    zsel = zmask & (ordv < t2) & (need > 0.0)
    out_ref[...] = jnp.where(pos | zsel, 1.0, 0.0)


def kernel(inputs, W):
    B = inputs.shape[0]
    x = inputs.reshape(B, -1)

    encoding = pl.pallas_call(
        _matmul_kernel,
        grid=(H // UBLK,),
        in_specs=[
            pl.BlockSpec((B, x.shape[1]), lambda i: (0, 0)),
            pl.BlockSpec((UBLK, x.shape[1]), lambda i: (i, 0)),
        ],
        out_specs=pl.BlockSpec((B, UBLK), lambda i: (0, i)),
        out_shape=jax.ShapeDtypeStruct((B, H), jnp.float32),
    )(x, W)

    enc3 = encoding.reshape(B, SUB, LANE)
    fired3 = pl.pallas_call(
        _scan_kernel,
        in_specs=[pl.BlockSpec(memory_space=pltpu.VMEM)],
        out_specs=pl.BlockSpec(memory_space=pltpu.VMEM),
        out_shape=jax.ShapeDtypeStruct((B, SUB, LANE), jnp.float32),
        scratch_shapes=[pltpu.VMEM((SUB, LANE), jnp.float32)],
    )(enc3)
    mask3 = pl.pallas_call(
        _final_kernel,
        grid=(B // RB,),
        in_specs=[
            pl.BlockSpec((RB, SUB, LANE), lambda i: (i, 0, 0)),
            pl.BlockSpec((RB, SUB, LANE), lambda i: (i, 0, 0)),
        ],
        out_specs=pl.BlockSpec((RB, SUB, LANE), lambda i: (i, 0, 0)),
        out_shape=jax.ShapeDtypeStruct((B, SUB, LANE), jnp.float32),
    )(enc3, fired3)
    return mask3.reshape(B, H)
